# Initial kernel scaffold; baseline (speedup 1.0000x reference)
import jax
import jax.numpy as jnp
from jax.experimental import pallas as pl


def kernel(probas, labels):
    raise NotImplementedError("write your pallas kernel here")


# speedup vs baseline: 15.6460x; 15.6460x over previous
import functools

import jax
import jax.numpy as jnp
from jax import lax
from jax.experimental import pallas as pl
from jax.experimental.pallas import tpu as pltpu
from jax.experimental.pallas import tpu_sc as plsc

C = 19
NB = 512
CB = C * NB
HIST = 2 * CB
NW = 32
N = 2 * 512 * 512
PPW = N // NW
CH = 1024
NCH = PPW // CH
HW = 512 * 512


def _sc_hist_body(p_hbm, l_hbm, cnt_out, sum_out, pbuf, lbuf, hcnt, hsum):
    cid = lax.axis_index("c")
    sid = lax.axis_index("s")
    wid = sid * 2 + cid
    img = wid // 16
    base = (wid % 16) * PPW

    zeros16 = jnp.zeros((16,), jnp.float32)

    def zero_body(i, _):
        hcnt[pl.ds(i * 16, 16)] = zeros16
        hsum[pl.ds(i * 16, 16)] = zeros16
        return 0

    lax.fori_loop(0, HIST // 16, zero_body, 0)

    c_lo = lax.iota(jnp.int32, 16)
    c_hi = c_lo + 16
    m_hi = c_hi < C
    c_hic = jnp.minimum(c_hi, C - 1)
    ones = jnp.ones((16,), jnp.float32)
    fscale = jnp.float32(NB)

    def chunk_body(t, _):
        off = base + t * CH
        pltpu.sync_copy(l_hbm.at[img, pl.ds(off, CH)], lbuf)
        pltpu.sync_copy(p_hbm.at[img, :, pl.ds(off, CH)], pbuf)

        def pix_body(i, _):
            bi = jnp.full((16,), i, jnp.int32)
            lab = plsc.load_gather(lbuf, [bi])
            fg = c_lo == lab
            p = plsc.load_gather(pbuf, [c_lo, bi])
            e = jnp.abs(jnp.where(fg, 1.0, 0.0).astype(jnp.float32) - p)
            bk = jnp.minimum((e * fscale).astype(jnp.int32), NB - 1)
            idx = jnp.where(fg, CB, 0) + c_lo * NB + bk
            plsc.addupdate_scatter(hcnt, [idx], ones)
            plsc.addupdate_scatter(hsum, [idx], e)
            fg2 = c_hi == lab
            p2 = plsc.load_gather(pbuf, [c_hic, bi])
            e2 = jnp.abs(jnp.where(fg2, 1.0, 0.0).astype(jnp.float32) - p2)
            bk2 = jnp.minimum((e2 * fscale).astype(jnp.int32), NB - 1)
            idx2 = jnp.where(fg2, CB, 0) + c_hic * NB + bk2
            plsc.addupdate_scatter(hcnt, [idx2], ones, mask=m_hi)
            plsc.addupdate_scatter(hsum, [idx2], e2, mask=m_hi)
            return 0

        lax.fori_loop(0, CH, pix_body, 0)
        return 0

    lax.fori_loop(0, NCH, chunk_body, 0)

    pltpu.sync_copy(hcnt, cnt_out.at[wid])
    pltpu.sync_copy(hsum, sum_out.at[wid])


_sc_hist = functools.partial(
    pl.kernel,
    out_type=(
        jax.ShapeDtypeStruct((NW, HIST), jnp.float32),
        jax.ShapeDtypeStruct((NW, HIST), jnp.float32),
    ),
    mesh=plsc.VectorSubcoreMesh(core_axis_name="c", subcore_axis_name="s"),
    compiler_params=pltpu.CompilerParams(needs_layout_passes=False),
    scratch_types=[
        pltpu.VMEM((C, CH), jnp.float32),
        pltpu.VMEM((CH,), jnp.int32),
        pltpu.VMEM((HIST,), jnp.float32),
        pltpu.VMEM((HIST,), jnp.float32),
    ],
)(_sc_hist_body)


def _cumsum_last(x):
    k = 1
    while k < x.shape[-1]:
        pad = jnp.zeros(x.shape[:-1] + (k,), x.dtype)
        x = x + jnp.concatenate([pad, x[..., :-k]], axis=-1)
        k *= 2
    return x


def _tc_finish_body(cnt_ref, sum_ref, out_ref):
    cnt = jnp.sum(cnt_ref[...], axis=0)
    sm = jnp.sum(sum_ref[...], axis=0)
    n0, n1 = cnt[0], cnt[1]
    s0, s1 = sm[0], sm[1]
    tot0 = jnp.sum(n0, axis=1, keepdims=True)
    tot1 = jnp.sum(n1, axis=1, keepdims=True)
    zb = tot0 - _cumsum_last(n0)
    pb = tot1 - _cumsum_last(n1)
    gts = tot1
    u0 = gts + zb
    inter = gts - pb - n1
    fg_term = s1 / jnp.maximum(u0, 1.0)
    bg_term = s0 * inter / jnp.maximum(u0 * (u0 + n0), 1.0)
    losses = jnp.sum(fg_term + bg_term, axis=1, keepdims=True)
    pres = (gts > 0.0).astype(jnp.float32)
    out_ref[0, 0] = jnp.sum(losses * pres) / jnp.maximum(jnp.sum(pres), 1.0)


def _tc_finish(cnt, sm):
    return pl.pallas_call(
        _tc_finish_body,
        out_shape=jax.ShapeDtypeStruct((1, 1), jnp.float32),
        out_specs=pl.BlockSpec(memory_space=pltpu.MemorySpace.SMEM),
    )(cnt, sm)


def kernel(probas, labels):
    p3 = probas.reshape(2, C, HW)
    l2 = labels.astype(jnp.int32).reshape(2, HW)
    cnt, sm = _sc_hist(p3, l2)
    cnt4 = cnt.reshape(NW, 2, C, NB)
    sm4 = sm.reshape(NW, 2, C, NB)
    return _tc_finish(cnt4, sm4)[0, 0]


# speedup vs baseline: 112.9029x; 7.2161x over previous
import functools

import jax
import jax.numpy as jnp
from jax import lax
from jax.experimental import pallas as pl
from jax.experimental.pallas import tpu as pltpu
from jax.experimental.pallas import tpu_sc as plsc

C = 19
NB = 512
CB = C * NB
HIST = 2 * CB
NW = 32
N = 2 * 512 * 512
PPW = N // NW
CH = 1024
NCH = PPW // CH
HW = 512 * 512


def _sc_hist_body(p_hbm, l_hbm, cnt_out, sum_out, pbuf, lbuf, hcnt, hsum):
    cid = lax.axis_index("c")
    sid = lax.axis_index("s")
    wid = sid * 2 + cid
    img = wid // 16
    base = (wid % 16) * PPW

    zeros16 = jnp.zeros((16,), jnp.float32)

    def zero_body(i, _):
        hcnt[pl.ds(i * 16, 16)] = zeros16
        hsum[pl.ds(i * 16, 16)] = zeros16
        return 0

    lax.fori_loop(0, HIST // 16, zero_body, 0)

    c_lo = lax.iota(jnp.int32, 16)
    c_hi = c_lo + 16
    m_hi = c_hi < C
    c_hic = jnp.minimum(c_hi, C - 1)
    base_lo = c_lo * NB
    base_hi = c_hic * NB
    ones = jnp.ones((16,), jnp.float32)
    fscale = jnp.float32(NB)

    def chunk_body(t, _):
        off = base + t * CH
        pltpu.sync_copy(l_hbm.at[img, pl.ds(off, CH)], lbuf)
        pltpu.sync_copy(p_hbm.at[img, :, pl.ds(off, CH)], pbuf)

        @functools.partial(plsc.parallel_loop, 0, CH, unroll=8)
        def pix_body(i):
            bi = jnp.full((16,), i, jnp.int32)
            lab = plsc.load_gather(lbuf, [bi])
            fg = c_lo == lab
            p = plsc.load_gather(pbuf, [c_lo, bi])
            e = jnp.where(fg, 1.0 - p, p)
            bk = jnp.minimum((e * fscale).astype(jnp.int32), NB - 1)
            idx = jnp.where(fg, CB, 0) + base_lo + bk
            plsc.addupdate_scatter(hcnt, [idx], ones)
            plsc.addupdate_scatter(hsum, [idx], e)
            fg2 = c_hi == lab
            p2 = plsc.load_gather(pbuf, [c_hic, bi])
            e2 = jnp.where(fg2, 1.0 - p2, p2)
            bk2 = jnp.minimum((e2 * fscale).astype(jnp.int32), NB - 1)
            idx2 = jnp.where(fg2, CB, 0) + base_hi + bk2
            plsc.addupdate_scatter(hcnt, [idx2], ones, mask=m_hi)
            plsc.addupdate_scatter(hsum, [idx2], e2, mask=m_hi)

        return 0

    lax.fori_loop(0, NCH, chunk_body, 0)

    pltpu.sync_copy(hcnt, cnt_out.at[wid])
    pltpu.sync_copy(hsum, sum_out.at[wid])


_sc_hist = functools.partial(
    pl.kernel,
    out_type=(
        jax.ShapeDtypeStruct((NW, HIST), jnp.float32),
        jax.ShapeDtypeStruct((NW, HIST), jnp.float32),
    ),
    mesh=plsc.VectorSubcoreMesh(core_axis_name="c", subcore_axis_name="s"),
    compiler_params=pltpu.CompilerParams(needs_layout_passes=False),
    scratch_types=[
        pltpu.VMEM((C, CH), jnp.float32),
        pltpu.VMEM((CH,), jnp.int32),
        pltpu.VMEM((HIST,), jnp.float32),
        pltpu.VMEM((HIST,), jnp.float32),
    ],
)(_sc_hist_body)


def _cumsum_last(x):
    k = 1
    while k < x.shape[-1]:
        pad = jnp.zeros(x.shape[:-1] + (k,), x.dtype)
        x = x + jnp.concatenate([pad, x[..., :-k]], axis=-1)
        k *= 2
    return x


def _tc_finish_body(cnt_ref, sum_ref, out_ref):
    cnt = jnp.sum(cnt_ref[...], axis=0)
    sm = jnp.sum(sum_ref[...], axis=0)
    n0, n1 = cnt[0], cnt[1]
    s0, s1 = sm[0], sm[1]
    tot0 = jnp.sum(n0, axis=1, keepdims=True)
    tot1 = jnp.sum(n1, axis=1, keepdims=True)
    zb = tot0 - _cumsum_last(n0)
    pb = tot1 - _cumsum_last(n1)
    gts = tot1
    u0 = gts + zb
    inter = gts - pb - n1
    fg_term = s1 / jnp.maximum(u0, 1.0)
    bg_term = s0 * inter / jnp.maximum(u0 * (u0 + n0), 1.0)
    losses = jnp.sum(fg_term + bg_term, axis=1, keepdims=True)
    pres = (gts > 0.0).astype(jnp.float32)
    out_ref[0, 0] = jnp.sum(losses * pres) / jnp.maximum(jnp.sum(pres), 1.0)


def _tc_finish(cnt, sm):
    return pl.pallas_call(
        _tc_finish_body,
        out_shape=jax.ShapeDtypeStruct((1, 1), jnp.float32),
        out_specs=pl.BlockSpec(memory_space=pltpu.MemorySpace.SMEM),
    )(cnt, sm)


def kernel(probas, labels):
    p3 = probas.reshape(2, C, HW)
    l2 = labels.astype(jnp.int32).reshape(2, HW)
    cnt, sm = _sc_hist(p3, l2)
    cnt4 = cnt.reshape(NW, 2, C, NB)
    sm4 = sm.reshape(NW, 2, C, NB)
    return _tc_finish(cnt4, sm4)[0, 0]
